# baseline (speedup 1.0000x reference)
import jax
import jax.numpy as jnp
from jax.experimental import pallas as pl
from jax.experimental.pallas import tpu as pltpu

_EPS = 1e-5
_NEG_SLOPE = 0.01

_H = 56
_C = 64
_F = 128
_KS = 3
_M = _H * _H
_PAD = 64


def _conv_positions(x_ref, w_ref):
    xt = x_ref[0]
    ow = jax.lax.broadcasted_iota(jnp.int32, (_M, _C), 0) % _H
    xl = jnp.where(ow == _H - 1, 0.0, xt)
    xr = jnp.where(ow == 0, 0.0, xt)
    zpad = jnp.zeros((_PAD, _C), jnp.float32)
    srcs = [jnp.concatenate([zpad, v, zpad], axis=0) for v in (xl, xt, xr)]
    acc = jnp.zeros((_M, _F), jnp.float32)
    for kh in range(_KS):
        for kw in range(_KS):
            off = _PAD + (kh - 1) * _H + (kw - 1)
            s = srcs[kw][off:off + _M, :]
            tap = kh * _KS + kw
            acc += jnp.dot(s, w_ref[tap * _C:(tap + 1) * _C, :],
                           preferred_element_type=jnp.float32)
    return acc


def _stats_kernel(x_ref, w_ref, stats_ref):
    y = _conv_positions(x_ref, w_ref)
    s = jnp.sum(y, axis=0).reshape(1, _F)
    ss = jnp.sum(y * y, axis=0).reshape(1, _F)
    stats_ref[...] = jnp.concatenate(
        [s, ss, jnp.zeros((6, _F), jnp.float32)], axis=0)


def _apply_kernel(x_ref, w_ref, scale_ref, shift_ref, o_ref):
    y = _conv_positions(x_ref, w_ref)
    o = y * scale_ref[...] + shift_ref[...]
    o = jnp.maximum(o, _NEG_SLOPE * o)
    o_ref[0] = o.T


@jax.jit
def _run(x, weight, gamma, beta):
    n = x.shape[0]
    m = n * _M

    x_flat = jnp.transpose(x, (0, 2, 3, 1)).reshape(n, _M, _C)

    w_mat = jnp.transpose(weight, (2, 3, 1, 0)).reshape(_KS * _KS * _C, _F)

    grid = (n,)
    parallel = pltpu.CompilerParams(dimension_semantics=("parallel",))

    stats = pl.pallas_call(
        _stats_kernel,
        out_shape=jax.ShapeDtypeStruct((n * 8, _F), jnp.float32),
        grid=grid,
        in_specs=[
            pl.BlockSpec((1, _M, _C), lambda i: (i, 0, 0)),
            pl.BlockSpec((_KS * _KS * _C, _F), lambda i: (0, 0)),
        ],
        out_specs=pl.BlockSpec((8, _F), lambda i: (i, 0)),
        compiler_params=parallel,
    )(x_flat, w_mat)

    stats = stats.reshape(n, 8, _F)
    inv_m = jnp.float32(1.0) / jnp.float32(m)
    mean = jnp.sum(stats[:, 0, :], axis=0) * inv_m
    var = jnp.maximum(jnp.sum(stats[:, 1, :], axis=0) * inv_m - mean * mean,
                      0.0)
    inv_std = jax.lax.rsqrt(var + _EPS)
    scale = (inv_std * gamma.astype(jnp.float32)).reshape(1, _F)
    shift = (beta.astype(jnp.float32) - mean * inv_std *
             gamma.astype(jnp.float32)).reshape(1, _F)

    out = pl.pallas_call(
        _apply_kernel,
        out_shape=jax.ShapeDtypeStruct((n, _F, _M), jnp.float32),
        grid=grid,
        in_specs=[
            pl.BlockSpec((1, _M, _C), lambda i: (i, 0, 0)),
            pl.BlockSpec((_KS * _KS * _C, _F), lambda i: (0, 0)),
            pl.BlockSpec((1, _F), lambda i: (0, 0)),
            pl.BlockSpec((1, _F), lambda i: (0, 0)),
        ],
        out_specs=pl.BlockSpec((1, _F, _M), lambda i: (i, 0, 0)),
        compiler_params=parallel,
    )(x_flat, w_mat, scale, shift)

    return out.reshape(n, _F, _H, _H)


def kernel(x, weight, bias, gamma, beta):
    del bias
    return _run(x, weight, gamma, beta)


# speedup vs baseline: 1.3037x; 1.3037x over previous
import jax
import jax.numpy as jnp
from jax.experimental import pallas as pl
from jax.experimental.pallas import tpu as pltpu

_EPS = 1e-5
_NEG_SLOPE = 0.01

_H = 56
_C = 64
_F = 128
_KS = 3
_M = _H * _H
_PAD = 64


def _conv_positions(x_ref, w_ref):
    xt = x_ref[0]
    ow = jax.lax.broadcasted_iota(jnp.int32, (_M, _C), 0) % _H
    xl = jnp.where(ow == _H - 1, 0.0, xt)
    xr = jnp.where(ow == 0, 0.0, xt)
    zpad = jnp.zeros((_PAD, _C), jnp.float32)
    srcs = [jnp.concatenate([zpad, v, zpad], axis=0) for v in (xl, xt, xr)]
    acc = jnp.zeros((_M, _F), jnp.float32)
    for kh in range(_KS):
        for kw in range(_KS):
            off = _PAD + (kh - 1) * _H + (kw - 1)
            s = srcs[kw][off:off + _M, :]
            tap = kh * _KS + kw
            acc += jnp.dot(s, w_ref[tap * _C:(tap + 1) * _C, :],
                           preferred_element_type=jnp.float32)
    return acc


def _stats_kernel(x_ref, w_ref, stats_ref):
    y = _conv_positions(x_ref, w_ref)
    s = jnp.sum(y, axis=0).reshape(1, _F)
    ss = jnp.sum(y * y, axis=0).reshape(1, _F)
    stats_ref[...] = jnp.concatenate(
        [s, ss, jnp.zeros((6, _F), jnp.float32)], axis=0)


def _apply_kernel(x_ref, w_ref, scale_ref, shift_ref, o_ref):
    y = _conv_positions(x_ref, w_ref)
    o = y * scale_ref[...] + shift_ref[...]
    o = jnp.maximum(o, _NEG_SLOPE * o)
    o_ref[0] = o


@jax.jit
def _run(x, weight, gamma, beta):
    n = x.shape[0]
    m = n * _M

    x_flat = jnp.transpose(x, (0, 2, 3, 1)).reshape(n, _M, _C)

    w_mat = jnp.transpose(weight, (2, 3, 1, 0)).reshape(_KS * _KS * _C, _F)

    grid = (n,)
    parallel = pltpu.CompilerParams(dimension_semantics=("parallel",))

    stats = pl.pallas_call(
        _stats_kernel,
        out_shape=jax.ShapeDtypeStruct((n * 8, _F), jnp.float32),
        grid=grid,
        in_specs=[
            pl.BlockSpec((1, _M, _C), lambda i: (i, 0, 0)),
            pl.BlockSpec((_KS * _KS * _C, _F), lambda i: (0, 0)),
        ],
        out_specs=pl.BlockSpec((8, _F), lambda i: (i, 0)),
        compiler_params=parallel,
    )(x_flat, w_mat)

    stats = stats.reshape(n, 8, _F)
    inv_m = jnp.float32(1.0) / jnp.float32(m)
    mean = jnp.sum(stats[:, 0, :], axis=0) * inv_m
    var = jnp.maximum(jnp.sum(stats[:, 1, :], axis=0) * inv_m - mean * mean,
                      0.0)
    inv_std = jax.lax.rsqrt(var + _EPS)
    scale = (inv_std * gamma.astype(jnp.float32)).reshape(1, _F)
    shift = (beta.astype(jnp.float32) - mean * inv_std *
             gamma.astype(jnp.float32)).reshape(1, _F)

    out = pl.pallas_call(
        _apply_kernel,
        out_shape=jax.ShapeDtypeStruct((n, _M, _F), jnp.float32),
        grid=grid,
        in_specs=[
            pl.BlockSpec((1, _M, _C), lambda i: (i, 0, 0)),
            pl.BlockSpec((_KS * _KS * _C, _F), lambda i: (0, 0)),
            pl.BlockSpec((1, _F), lambda i: (0, 0)),
            pl.BlockSpec((1, _F), lambda i: (0, 0)),
        ],
        out_specs=pl.BlockSpec((1, _M, _F), lambda i: (i, 0, 0)),
        compiler_params=parallel,
    )(x_flat, w_mat, scale, shift)

    out = out.reshape(n, _H, _H, _F)
    return jnp.transpose(out, (0, 3, 1, 2))


def kernel(x, weight, bias, gamma, beta):
    del bias
    return _run(x, weight, gamma, beta)


# speedup vs baseline: 1.4524x; 1.1140x over previous
import jax
import jax.numpy as jnp
from jax.experimental import pallas as pl
from jax.experimental.pallas import tpu as pltpu

_EPS = 1e-5
_NEG_SLOPE = 0.01

_H = 56
_C = 64
_F = 128
_KS = 3
_M = _H * _H
_PAD = 64


def _conv_positions(x_ref, w_ref):
    xt = x_ref[0]
    ow = jax.lax.broadcasted_iota(jnp.int32, (_M, _C), 0) % _H
    xl = jnp.where(ow == _H - 1, 0.0, xt)
    xr = jnp.where(ow == 0, 0.0, xt)
    zpad = jnp.zeros((_PAD, _C), jnp.float32)
    srcs = [jnp.concatenate([zpad, v, zpad], axis=0) for v in (xl, xt, xr)]
    acc = jnp.zeros((_M, _F), jnp.float32)
    for kw in range(_KS):
        off0 = _PAD + (kw - 1)
        scat = jnp.concatenate(
            [srcs[kw][off0 + (kh - 1) * _H:off0 + (kh - 1) * _H + _M, :]
             for kh in range(_KS)], axis=1)
        acc += jnp.dot(scat, w_ref[kw * _KS * _C:(kw + 1) * _KS * _C, :],
                       preferred_element_type=jnp.float32)
    return acc


def _stats_kernel(x_ref, w_ref, stats_ref):
    y = _conv_positions(x_ref, w_ref)
    s = jnp.sum(y, axis=0).reshape(1, _F)
    ss = jnp.sum(y * y, axis=0).reshape(1, _F)
    stats_ref[...] = jnp.concatenate(
        [s, ss, jnp.zeros((6, _F), jnp.float32)], axis=0)


def _apply_kernel(x_ref, w_ref, scale_ref, shift_ref, o_ref):
    y = _conv_positions(x_ref, w_ref)
    o = y * scale_ref[...] + shift_ref[...]
    o = jnp.maximum(o, _NEG_SLOPE * o)
    o_ref[0] = o


@jax.jit
def _run(x, weight, gamma, beta):
    n = x.shape[0]
    m = n * _M

    x_flat = jnp.transpose(x, (0, 2, 3, 1)).reshape(n, _M, _C)

    w_mat = jnp.transpose(weight, (3, 2, 1, 0)).reshape(_KS * _KS * _C, _F)

    grid = (n,)
    parallel = pltpu.CompilerParams(dimension_semantics=("parallel",))

    stats = pl.pallas_call(
        _stats_kernel,
        out_shape=jax.ShapeDtypeStruct((n * 8, _F), jnp.float32),
        grid=grid,
        in_specs=[
            pl.BlockSpec((1, _M, _C), lambda i: (i, 0, 0)),
            pl.BlockSpec((_KS * _KS * _C, _F), lambda i: (0, 0)),
        ],
        out_specs=pl.BlockSpec((8, _F), lambda i: (i, 0)),
        compiler_params=parallel,
    )(x_flat, w_mat)

    stats = stats.reshape(n, 8, _F)
    inv_m = jnp.float32(1.0) / jnp.float32(m)
    mean = jnp.sum(stats[:, 0, :], axis=0) * inv_m
    var = jnp.maximum(jnp.sum(stats[:, 1, :], axis=0) * inv_m - mean * mean,
                      0.0)
    inv_std = jax.lax.rsqrt(var + _EPS)
    scale = (inv_std * gamma.astype(jnp.float32)).reshape(1, _F)
    shift = (beta.astype(jnp.float32) - mean * inv_std *
             gamma.astype(jnp.float32)).reshape(1, _F)

    out = pl.pallas_call(
        _apply_kernel,
        out_shape=jax.ShapeDtypeStruct((n, _M, _F), jnp.float32),
        grid=grid,
        in_specs=[
            pl.BlockSpec((1, _M, _C), lambda i: (i, 0, 0)),
            pl.BlockSpec((_KS * _KS * _C, _F), lambda i: (0, 0)),
            pl.BlockSpec((1, _F), lambda i: (0, 0)),
            pl.BlockSpec((1, _F), lambda i: (0, 0)),
        ],
        out_specs=pl.BlockSpec((1, _M, _F), lambda i: (i, 0, 0)),
        compiler_params=parallel,
    )(x_flat, w_mat, scale, shift)

    out = out.reshape(n, _H, _H, _F)
    return jnp.transpose(out, (0, 3, 1, 2))


def kernel(x, weight, bias, gamma, beta):
    del bias
    return _run(x, weight, gamma, beta)
